# baseline (speedup 1.0000x reference)
import jax
import jax.numpy as jnp
from jax import lax
from jax.experimental import pallas as pl
from jax.experimental.pallas import tpu as pltpu
from jax.experimental.pallas import tpu_sc as plsc

B, D, H, W, M = 32, 4, 128, 128, 128
HW = H * W
L = 16
NS = 16
BPT = B // NS
NJ = BPT * M // L


def _take16(x, idx):
    dn = lax.GatherDimensionNumbers(
        offset_dims=(), collapsed_slice_dims=(0,), start_index_map=(0,))
    return lax.gather(x, idx[:, None], dn, slice_sizes=(1,),
                      mode=lax.GatherScatterMode.PROMISE_IN_BOUNDS)


def _sc_body(out_hbm, mask_hbm, ind_hbm, targ_hbm, res_hbm,
             ind_v, mask_v, targ_v, idx_v, pred_v, part_v, outv,
             shared, sem_i, sem_t, *sem_g):
    sid = lax.axis_index("s")
    lane = lax.iota(jnp.int32, L)

    cp_ind = pltpu.async_copy(
        ind_hbm.at[pl.ds(sid * BPT * M, BPT * M)], ind_v, sem_i)
    cp_mask = pltpu.async_copy(
        mask_hbm.at[pl.ds(sid * BPT * M, BPT * M)], mask_v, sem_t)
    cp_targ = pltpu.async_copy(
        targ_hbm.at[pl.ds(sid * BPT * M * D, BPT * M * D)], targ_v, sem_t)

    @pl.when(sid == 0)
    def _init():
        part_v[...] = jnp.zeros((L,), jnp.float32)
        pltpu.sync_copy(part_v, shared)
    plsc.subcore_barrier()

    cp_ind.wait()

    lq = lane >> 2
    ld = lane & (D - 1)
    sels = [q * 4 + lq for q in range(4)]
    dbase = ld * HW

    def idx_body(j, _):
        iv = ind_v[pl.ds(j * L, L)]
        bl = j // (M // L)
        base = (sid * BPT + bl) * (D * HW) + dbase
        for q in range(4):
            ivq = _take16(iv, sels[q])
            idx_v[pl.ds((j * 4 + q) * L, L)] = base + ivq
        return 0

    NR = BPT * D

    def fire(r, sem):
        pltpu.async_copy(out_hbm.at[idx_v.at[pl.ds(r * M, M)]],
                         pred_v.at[pl.ds(r * M, M)], sem)
        return 0

    NG = len(sem_g)
    for g in range(NG):
        lax.fori_loop(g * NJ // NG, (g + 1) * NJ // NG, idx_body, 0)
        lax.fori_loop(g * NR // NG, (g + 1) * NR // NG,
                      lambda r, _, s=sem_g[g]: fire(r, s), 0)

    def sum_body(j, carry):
        acc, accm = carry
        mvi = mask_v[pl.ds(j * L, L)].astype(jnp.float32)
        accm = accm + mvi
        for q in range(4):
            mv = _take16(mvi, sels[q])
            pv = pred_v[pl.ds((j * 4 + q) * L, L)]
            tv = targ_v[pl.ds((j * 4 + q) * L, L)]
            acc = acc + mv * jnp.abs(pv - tv)
        return acc, accm

    cp_mask.wait()
    cp_targ.wait()
    quarter = BPT * M * D // NG
    zero = jnp.zeros((L,), jnp.float32)
    carry = (zero, zero)
    for g in range(NG):
        pltpu.make_async_copy(
            out_hbm.at[pl.ds(0, quarter)],
            pred_v.at[pl.ds(g * quarter, quarter)], sem_g[g]).wait()
        carry = lax.fori_loop(g * NJ // NG, (g + 1) * NJ // NG,
                              sum_body, carry)
    acc, accm = carry

    y = acc + _take16(acc, lane ^ 4)
    y = y + _take16(y, lane ^ 8)
    for sh in (1, 2, 4, 8):
        accm = accm + _take16(accm, lane ^ sh)
    part = jnp.where(lane < D, y, 0.0)
    part = jnp.where(lane == D, accm, part)
    part_v[...] = part
    pltpu.sync_copy(part_v, shared.at[lane], add=True)

    plsc.subcore_barrier()

    @pl.when(sid == 0)
    def _final():
        pltpu.sync_copy(shared, outv)
        tot = outv[...]
        msum = _take16(tot, jnp.full((L,), D, jnp.int32))
        outv[...] = jnp.where(lane < D, tot, 0.0) / (msum + 1e-4)
        pltpu.sync_copy(outv.at[pl.ds(0, D)], res_hbm)


_sc_call = pl.kernel(
    _sc_body,
    out_type=jax.ShapeDtypeStruct((D,), jnp.float32),
    mesh=plsc.VectorSubcoreMesh(
        core_axis_name="c", subcore_axis_name="s", num_cores=1),
    scratch_types=[
        pltpu.VMEM((BPT * M,), jnp.int32),
        pltpu.VMEM((BPT * M,), jnp.int32),
        pltpu.VMEM((BPT * M * D,), jnp.float32),
        pltpu.VMEM((BPT * M * D,), jnp.int32),
        pltpu.VMEM((BPT * M * D,), jnp.float32),
        pltpu.VMEM((L,), jnp.float32),
        pltpu.VMEM((L,), jnp.float32),
        pltpu.VMEM_SHARED((L,), jnp.float32),
        pltpu.SemaphoreType.DMA,
        pltpu.SemaphoreType.DMA,
        pltpu.SemaphoreType.DMA,
        pltpu.SemaphoreType.DMA,
        pltpu.SemaphoreType.DMA,
        pltpu.SemaphoreType.DMA,
    ],
)


def kernel(output, mask, ind, target):
    return _sc_call(output.reshape(-1), mask.reshape(-1), ind.reshape(-1),
                    target.reshape(-1))
